# Initial kernel scaffold; baseline (speedup 1.0000x reference)
import jax
import jax.numpy as jnp
from jax.experimental import pallas as pl


def kernel(edge_row, edge_col, edge_val, features, W1, b1, W2, b2):
    raise NotImplementedError("write your pallas kernel here")


# speedup vs baseline: 1.0020x; 1.0020x over previous
import jax
import jax.numpy as jnp
from jax.experimental import pallas as pl


def _epilogue_body(lx_ref, x_ref, w1t_ref, w2t_ref, b_ref, o_ref):
    lx = lx_ref[...]
    x = x_ref[...]
    a = lx + x
    m = lx * x
    o_ref[...] = (
        jnp.dot(a, w1t_ref[...], preferred_element_type=jnp.float32)
        + jnp.dot(m, w2t_ref[...], preferred_element_type=jnp.float32)
        + b_ref[:1, :]
    )


def _epilogue(lx, features, W1, b1, W2, b2):
    n, d = features.shape
    w1t = W1.T
    w2t = W2.T
    bias = jnp.broadcast_to((b1 + b2)[None, :], (8, d))
    BLK = 2048
    return pl.pallas_call(
        _epilogue_body,
        grid=(n // BLK,),
        in_specs=[
            pl.BlockSpec((BLK, d), lambda i: (i, 0)),
            pl.BlockSpec((BLK, d), lambda i: (i, 0)),
            pl.BlockSpec((d, d), lambda i: (0, 0)),
            pl.BlockSpec((d, d), lambda i: (0, 0)),
            pl.BlockSpec((8, d), lambda i: (0, 0)),
        ],
        out_specs=pl.BlockSpec((BLK, d), lambda i: (i, 0)),
        out_shape=jax.ShapeDtypeStruct((n, d), jnp.float32),
    )(lx, features, w1t, w2t, bias)


def kernel(edge_row, edge_col, edge_val, features, W1, b1, W2, b2):
    n = features.shape[0]
    gathered = edge_val[:, None] * jnp.take(features, edge_col, axis=0)
    lx = jax.ops.segment_sum(gathered, edge_row, num_segments=n)
    return _epilogue(lx, features, W1, b1, W2, b2)


# speedup vs baseline: 6.9880x; 6.9743x over previous
import functools

import jax
import jax.numpy as jnp
from jax import lax
from jax.experimental import pallas as pl
from jax.experimental.pallas import tpu as pltpu
from jax.experimental.pallas import tpu_sc as plsc

QR = 16384
ACC_ROWS = 16512
C = 4096
G = 128
CB = C + G
ZROWS = 258


def _lane(v, l):
    return lax.squeeze(lax.slice(v, (l,), (l + 1,)), (0,))


def _make_spmm(n, d, nnz):
    mesh = plsc.VectorSubcoreMesh(core_axis_name="c", subcore_axis_name="s")
    epc = nnz // 16
    nch = epc // C

    @functools.partial(
        pl.kernel,
        mesh=mesh,
        out_type=jax.ShapeDtypeStruct((n, d), jnp.float32),
        compiler_params=pltpu.CompilerParams(
            needs_layout_passes=False, use_tc_tiling_on_sc=False),
        scratch_types=[
            pltpu.VMEM((C,), jnp.int32),
            pltpu.VMEM((C,), jnp.int32),
            pltpu.VMEM((C,), jnp.float32),
            pltpu.VMEM((CB,), jnp.int32),
            pltpu.VMEM((CB,), jnp.int32),
            pltpu.VMEM((CB,), jnp.float32),
            pltpu.VMEM((G,), jnp.int32),
            pltpu.VMEM((G, 64), jnp.float32),
            pltpu.VMEM((ZROWS, 64), jnp.float32),
            pltpu.VMEM_SHARED((ACC_ROWS, 64), jnp.float32),
            pltpu.SemaphoreType.DMA,
        ],
    )
    def spmm(row_hbm, col_hbm, val_hbm, feat_hbm, out_hbm,
             row_b, col_b, val_b, colc, lrowc, valc, lrow_batch, gbuf, zbuf,
             acc, sem):
        cid = lax.axis_index("c")
        sid = lax.axis_index("s")
        iota = lax.iota(jnp.int32, 16)
        zvec = jnp.zeros((16,), jnp.float32)

        def zb(i, carry):
            for k in range(4):
                zbuf[i, pl.ds(16 * k, 16)] = zvec
            return carry
        lax.fori_loop(0, ZROWS, zb, 0)

        for q in range(2):
            qid = 2 * cid + q
            lo = qid * QR

            for z in range(4):
                pltpu.sync_copy(zbuf, acc.at[pl.ds(sid * 1032 + z * ZROWS, ZROWS)])
            plsc.subcore_barrier()

            def chunk_body(ch, carry):
                base = sid * epc + ch * C
                pltpu.sync_copy(row_hbm.at[pl.ds(base, C)], row_b)
                pltpu.sync_copy(col_hbm.at[pl.ds(base, C)], col_b)
                pltpu.sync_copy(val_hbm.at[pl.ds(base, C)], val_b)

                def comp(i, cnt):
                    r = row_b[pl.ds(i * 16, 16)]
                    cc = col_b[pl.ds(i * 16, 16)]
                    vv = val_b[pl.ds(i * 16, 16)]
                    lr = r - jnp.full((16,), lo, jnp.int32)
                    m = (lr >= jnp.zeros((16,), jnp.int32)) & (
                        lr < jnp.full((16,), QR, jnp.int32))
                    mi = m.astype(jnp.int32)
                    cs = plsc.cumsum(mi)
                    pos = cs - mi + jnp.full((16,), cnt, jnp.int32)
                    plsc.store_scatter(colc, [pos], cc, mask=m)
                    plsc.store_scatter(lrowc, [pos], lr, mask=m)
                    plsc.store_scatter(valc, [pos], vv, mask=m)
                    return cnt + _lane(cs, 15)
                cnt = lax.fori_loop(0, C // 16, comp, jnp.int32(0))

                for j in range(G // 16):
                    colc[pl.ds(cnt + j * 16, 16)] = iota + (16 * j)
                    lrowc[pl.ds(cnt + j * 16, 16)] = iota + (16 * j + QR)
                    valc[pl.ds(cnt + j * 16, 16)] = zvec

                nb = (cnt + (G - 1)) // G

                def batch_body(g, carry2):
                    off = g * G
                    for j in range(G // 16):
                        lrow_batch[pl.ds(j * 16, 16)] = lrowc[pl.ds(off + j * 16, 16)]
                    pltpu.async_copy(feat_hbm.at[colc.at[pl.ds(off, G)]],
                                     gbuf, sem).wait()

                    def scale(e16, carry3):
                        vv = valc[pl.ds(off + e16 * 16, 16)]
                        for l in range(16):
                            sv = _lane(vv, l)
                            row = e16 * 16 + l
                            sv_v = jnp.full((16,), sv, jnp.float32)
                            for k in range(4):
                                gbuf[row, pl.ds(16 * k, 16)] = (
                                    gbuf[row, pl.ds(16 * k, 16)] * sv_v)
                        return carry3
                    lax.fori_loop(0, G // 16, scale, 0)

                    pltpu.sync_copy(gbuf, acc.at[lrow_batch], add=True)
                    return carry2
                lax.fori_loop(0, nb, batch_body, 0)
                return carry
            lax.fori_loop(0, nch, chunk_body, 0)

            plsc.subcore_barrier()
            pltpu.sync_copy(acc.at[pl.ds(sid * 1024, 1024)],
                            out_hbm.at[pl.ds(lo + sid * 1024, 1024)])
            plsc.subcore_barrier()

    return spmm


def _epilogue_body(lx_ref, x_ref, w1t_ref, w2t_ref, b_ref, o_ref):
    lx = lx_ref[...]
    x = x_ref[...]
    a = lx + x
    m = lx * x
    o_ref[...] = (
        jnp.dot(a, w1t_ref[...], preferred_element_type=jnp.float32)
        + jnp.dot(m, w2t_ref[...], preferred_element_type=jnp.float32)
        + b_ref[:1, :]
    )


def _epilogue(lx, features, W1, b1, W2, b2):
    n, d = features.shape
    w1t = W1.T
    w2t = W2.T
    bias = jnp.broadcast_to((b1 + b2)[None, :], (8, d))
    BLK = 2048
    return pl.pallas_call(
        _epilogue_body,
        grid=(n // BLK,),
        in_specs=[
            pl.BlockSpec((BLK, d), lambda i: (i, 0)),
            pl.BlockSpec((BLK, d), lambda i: (i, 0)),
            pl.BlockSpec((d, d), lambda i: (0, 0)),
            pl.BlockSpec((d, d), lambda i: (0, 0)),
            pl.BlockSpec((8, d), lambda i: (0, 0)),
        ],
        out_specs=pl.BlockSpec((BLK, d), lambda i: (i, 0)),
        out_shape=jax.ShapeDtypeStruct((n, d), jnp.float32),
    )(lx, features, w1t, w2t, bias)


def kernel(edge_row, edge_col, edge_val, features, W1, b1, W2, b2):
    n, d = features.shape
    nnz = edge_row.shape[0]
    er = edge_row.astype(jnp.int32)
    ec = edge_col.astype(jnp.int32)
    lx = _make_spmm(n, d, nnz)(er, ec, edge_val, features)
    return _epilogue(lx, features, W1, b1, W2, b2)


# speedup vs baseline: 15.1064x; 2.1617x over previous
import functools

import jax
import jax.numpy as jnp
from jax import lax
from jax.experimental import pallas as pl
from jax.experimental.pallas import tpu as pltpu
from jax.experimental.pallas import tpu_sc as plsc

QR = 16384
ACC_ROWS = 16512
C = 4096
G = 128
CB = C + G
ZROWS = 129


def _lane(v, l):
    return lax.squeeze(lax.slice(v, (l,), (l + 1,)), (0,))


def _make_spmm(n, d, nnz):
    mesh = plsc.VectorSubcoreMesh(core_axis_name="c", subcore_axis_name="s")
    epc = nnz // 16
    nch = epc // C

    @functools.partial(
        pl.kernel,
        mesh=mesh,
        out_type=jax.ShapeDtypeStruct((n, d), jnp.float32),
        compiler_params=pltpu.CompilerParams(
            needs_layout_passes=False, use_tc_tiling_on_sc=False),
        scratch_types=[
            pltpu.VMEM((C,), jnp.int32),
            pltpu.VMEM((C,), jnp.int32),
            pltpu.VMEM((C,), jnp.float32),
            pltpu.VMEM((CB,), jnp.int32),
            pltpu.VMEM((CB,), jnp.int32),
            pltpu.VMEM((CB,), jnp.float32),
            pltpu.VMEM((G,), jnp.int32),
            pltpu.VMEM((G,), jnp.int32),
            pltpu.VMEM((G, 64), jnp.float32),
            pltpu.VMEM((G, 64), jnp.float32),
            pltpu.VMEM((ZROWS, 64), jnp.float32),
            pltpu.VMEM_SHARED((ACC_ROWS, 64), jnp.float32),
            pltpu.SemaphoreType.DMA,
            pltpu.SemaphoreType.DMA,
            pltpu.SemaphoreType.DMA,
            pltpu.SemaphoreType.DMA,
        ],
    )
    def spmm(row_hbm, col_hbm, val_hbm, feat_hbm, out_hbm,
             row_b, col_b, val_b, colc, lrowc, valc,
             lr0, lr1, gbuf0, gbuf1, zbuf,
             acc, gsem0, gsem1, ssem0, ssem1):
        cid = lax.axis_index("c")
        sid = lax.axis_index("s")
        iota = lax.iota(jnp.int32, 16)
        zvec = jnp.zeros((16,), jnp.float32)

        def zb(i, carry):
            for k in range(4):
                zbuf[i, pl.ds(16 * k, 16)] = zvec
            return carry
        lax.fori_loop(0, ZROWS, zb, 0)

        def qpass(q, qcarry):
            qid = 2 * cid + q
            lo = qid * QR

            for z in range(8):
                pltpu.sync_copy(zbuf, acc.at[pl.ds(sid * 1032 + z * ZROWS, ZROWS)])
            plsc.subcore_barrier()

            def chunk_body(ch, carry):
                base = sid * epc + ch * C
                pltpu.sync_copy(row_hbm.at[pl.ds(base, C)], row_b)
                pltpu.sync_copy(col_hbm.at[pl.ds(base, C)], col_b)
                pltpu.sync_copy(val_hbm.at[pl.ds(base, C)], val_b)

                def comp(i, cnt_v):
                    r = row_b[pl.ds(i * 16, 16)]
                    cc = col_b[pl.ds(i * 16, 16)]
                    vv = val_b[pl.ds(i * 16, 16)]
                    lr = r - jnp.full((16,), lo, jnp.int32)
                    m = (lr >= jnp.zeros((16,), jnp.int32)) & (
                        lr < jnp.full((16,), QR, jnp.int32))
                    mi = m.astype(jnp.int32)
                    cs = plsc.cumsum(mi)
                    pos = cs - mi + cnt_v
                    plsc.store_scatter(colc, [pos], cc, mask=m)
                    plsc.store_scatter(lrowc, [pos], lr, mask=m)
                    plsc.store_scatter(valc, [pos], vv, mask=m)
                    return cnt_v + plsc.all_reduce_population_count(m)
                cnt_v = lax.fori_loop(0, C // 16, comp,
                                      jnp.zeros((16,), jnp.int32))
                cnt = _lane(cnt_v, 0)

                for j in range(G // 16):
                    colc[pl.ds(cnt + j * 16, 16)] = iota + (16 * j)
                    lrowc[pl.ds(cnt + j * 16, 16)] = iota + (16 * j + QR)
                    valc[pl.ds(cnt + j * 16, 16)] = zvec

                nb = (cnt + (G - 1)) // G

                def gdesc(g, gb, gs):
                    return pltpu.make_async_copy(
                        feat_hbm.at[colc.at[pl.ds(g * G, G)]], gb, gs)

                def sdesc(gb, lr, ss):
                    return pltpu.make_async_copy(gb, acc.at[lr], ss)

                def stage(g, lr):
                    off = g * G
                    for j in range(G // 16):
                        lr[pl.ds(j * 16, 16)] = lrowc[pl.ds(off + j * 16, 16)]

                def scale(gb, off):
                    def scale16(e16, carry3):
                        vv = valc[pl.ds(off + e16 * 16, 16)]
                        for l in range(16):
                            sv = _lane(vv, l)
                            row = e16 * 16 + l
                            sv_v = jnp.full((16,), sv, jnp.float32)
                            for k in range(4):
                                gb[row, pl.ds(16 * k, 16)] = (
                                    gb[row, pl.ds(16 * k, 16)] * sv_v)
                        return carry3
                    lax.fori_loop(0, G // 16, scale16, 0)

                bufs = ((gbuf0, gsem0, ssem0, lr0),
                        (gbuf1, gsem1, ssem1, lr1))

                @pl.when(nb >= 1)
                def _prologue():
                    stage(0, lr0)
                    gdesc(0, gbuf0, gsem0).start()

                def process(g, p):
                    gb, gs, ss, lr = bufs[p]
                    gbq, gsq, ssq, lrq = bufs[1 - p]
                    gdesc(g, gb, gs).wait()

                    @pl.when(g + 1 < nb)
                    def _issue_next():
                        @pl.when(g >= 1)
                        def _wait_prev_scatter():
                            sdesc(gbq, lrq, ssq).wait()
                        stage(g + 1, lrq)
                        gdesc(g + 1, gbq, gsq).start()

                    scale(gb, g * G)
                    sdesc(gb, lr, ss).start(add=True)

                def batch_body(g, carry2):
                    even = (g % 2) == 0

                    @pl.when(even)
                    def _e():
                        process(g, 0)

                    @pl.when(jnp.logical_not(even))
                    def _o():
                        process(g, 1)
                    return carry2
                lax.fori_loop(0, nb, batch_body, 0)

                def drain(p):
                    gb, gs, ss, lr = bufs[p]
                    sdesc(gb, lr, ss).wait()

                @pl.when(nb >= 2)
                def _drain_prev():
                    @pl.when((nb - 2) % 2 == 0)
                    def _d0():
                        drain(0)

                    @pl.when((nb - 2) % 2 == 1)
                    def _d1():
                        drain(1)

                @pl.when(nb >= 1)
                def _drain_last():
                    @pl.when((nb - 1) % 2 == 0)
                    def _d0():
                        drain(0)

                    @pl.when((nb - 1) % 2 == 1)
                    def _d1():
                        drain(1)
                return carry
            lax.fori_loop(0, nch, chunk_body, 0)

            plsc.subcore_barrier()
            pltpu.sync_copy(acc.at[pl.ds(sid * 1024, 1024)],
                            out_hbm.at[pl.ds(lo + sid * 1024, 1024)])
            plsc.subcore_barrier()
            return qcarry
        lax.fori_loop(0, 2, qpass, 0)

    return spmm


def _epilogue_body(lx_ref, x_ref, w1t_ref, w2t_ref, b_ref, o_ref):
    lx = lx_ref[...]
    x = x_ref[...]
    a = lx + x
    m = lx * x
    o_ref[...] = (
        jnp.dot(a, w1t_ref[...], preferred_element_type=jnp.float32)
        + jnp.dot(m, w2t_ref[...], preferred_element_type=jnp.float32)
        + b_ref[:1, :]
    )


def _epilogue(lx, features, W1, b1, W2, b2):
    n, d = features.shape
    w1t = W1.T
    w2t = W2.T
    bias = jnp.broadcast_to((b1 + b2)[None, :], (8, d))
    BLK = 2048
    return pl.pallas_call(
        _epilogue_body,
        grid=(n // BLK,),
        in_specs=[
            pl.BlockSpec((BLK, d), lambda i: (i, 0)),
            pl.BlockSpec((BLK, d), lambda i: (i, 0)),
            pl.BlockSpec((d, d), lambda i: (0, 0)),
            pl.BlockSpec((d, d), lambda i: (0, 0)),
            pl.BlockSpec((8, d), lambda i: (0, 0)),
        ],
        out_specs=pl.BlockSpec((BLK, d), lambda i: (i, 0)),
        out_shape=jax.ShapeDtypeStruct((n, d), jnp.float32),
    )(lx, features, w1t, w2t, bias)


def kernel(edge_row, edge_col, edge_val, features, W1, b1, W2, b2):
    n, d = features.shape
    nnz = edge_row.shape[0]
    er = edge_row.astype(jnp.int32)
    ec = edge_col.astype(jnp.int32)
    lx = _make_spmm(n, d, nnz)(er, ec, edge_val, features)
    return _epilogue(lx, features, W1, b1, W2, b2)
